# baseline (speedup 1.0000x reference)
import functools

import jax
import jax.numpy as jnp
from jax import lax
from jax.experimental import pallas as pl
from jax.experimental.pallas import tpu as pltpu
from jax.experimental.pallas import tpu_sc as plsc

N_NODES = 10000
D_FEAT = 128
N_EDGES = 320000

NC = 2
NS = 16
NW = NC * NS
L = 16

CH = 128
NCHUNK = 79
EPW = NCHUNK * CH
E_PAD = NW * EPW

DRAIN = 624
DRAIN_TAIL = N_NODES - NS * DRAIN


def _combine_body(w_ref, x_ref, h_ref, o_ref):
    o_ref[...] = x_ref[...] * w_ref[0] + h_ref[...] * w_ref[1]


def _finish_body(b_ref, p_ref, o_ref):
    s = p_ref[0] + p_ref[1] + b_ref[0]
    o_ref[...] = jax.nn.sigmoid(s)


def _sc_body(y_hbm, src_hbm, dst_hbm, w_hbm, zeros_hbm, out_hbm,
             src_v, dst_v, w_v, rows_v, agg_sh, sem):
    cid = lax.axis_index("c")
    sid = lax.axis_index("s")
    wid = cid * NS + sid

    @pl.when(sid == 0)
    def _():
        pltpu.sync_copy(zeros_hbm, agg_sh)

    plsc.subcore_barrier()

    def chunk_body(c, carry):
        base = wid * EPW + c * CH
        pltpu.sync_copy(src_hbm.at[pl.ds(base, CH)], src_v)
        pltpu.sync_copy(dst_hbm.at[pl.ds(base, CH)], dst_v)
        pltpu.sync_copy(w_hbm.at[pl.ds(base, CH)], w_v)
        pltpu.async_copy(y_hbm.at[src_v], rows_v, sem).wait()

        def edge_body(i, carry2):
            wspl = plsc.load_gather(w_v, [jnp.full((L,), i, jnp.int32)])
            for j in range(D_FEAT // L):
                rows_v[i, pl.ds(j * L, L)] = rows_v[i, pl.ds(j * L, L)] * wspl
            return carry2

        lax.fori_loop(0, CH, edge_body, 0, unroll=4)
        pltpu.sync_copy(rows_v, agg_sh.at[dst_v], add=True)
        return carry

    lax.fori_loop(0, NCHUNK, chunk_body, 0)

    plsc.subcore_barrier()
    pltpu.sync_copy(agg_sh.at[pl.ds(sid * DRAIN, DRAIN)],
                    out_hbm.at[cid, pl.ds(sid * DRAIN, DRAIN)])

    @pl.when(sid == 0)
    def _():
        pltpu.sync_copy(agg_sh.at[pl.ds(NS * DRAIN, DRAIN_TAIL)],
                        out_hbm.at[cid, pl.ds(NS * DRAIN, DRAIN_TAIL)])


_sc_scatter = functools.partial(
    pl.kernel,
    out_type=jax.ShapeDtypeStruct((NC, N_NODES, D_FEAT), jnp.float32),
    mesh=plsc.VectorSubcoreMesh(core_axis_name="c", subcore_axis_name="s",
                                num_cores=NC, num_subcores=NS),
    compiler_params=pltpu.CompilerParams(needs_layout_passes=False),
    scratch_types=[
        pltpu.VMEM((CH,), jnp.int32),
        pltpu.VMEM((CH,), jnp.int32),
        pltpu.VMEM((CH,), jnp.float32),
        pltpu.VMEM((CH, D_FEAT), jnp.float32),
        pltpu.VMEM_SHARED((N_NODES, D_FEAT), jnp.float32),
        pltpu.SemaphoreType.DMA,
    ],
)(_sc_body)


@jax.jit
def kernel(x, hint_matrix, edge_index, edge_weights, W, b):
    xs = x[0]
    hs = hint_matrix[0]
    wv = W[:, 0]
    pad = E_PAD - N_EDGES
    src = jnp.concatenate([edge_index[0], jnp.zeros((pad,), jnp.int32)])
    dst = jnp.concatenate([edge_index[1], jnp.zeros((pad,), jnp.int32)])
    edge_weights = jnp.concatenate(
        [edge_weights, jnp.zeros((pad,), jnp.float32)])

    rb = 1000
    y = pl.pallas_call(
        _combine_body,
        grid=(N_NODES // rb,),
        in_specs=[
            pl.BlockSpec(memory_space=pltpu.SMEM),
            pl.BlockSpec((rb, D_FEAT), lambda i: (i, 0)),
            pl.BlockSpec((rb, D_FEAT), lambda i: (i, 0)),
        ],
        out_specs=pl.BlockSpec((rb, D_FEAT), lambda i: (i, 0)),
        out_shape=jax.ShapeDtypeStruct((N_NODES, D_FEAT), jnp.float32),
    )(wv, xs, hs)

    zeros = jnp.zeros((N_NODES, D_FEAT), jnp.float32)
    partials = _sc_scatter(y, src, dst, edge_weights, zeros)

    out = pl.pallas_call(
        _finish_body,
        grid=(N_NODES // rb,),
        in_specs=[
            pl.BlockSpec(memory_space=pltpu.SMEM),
            pl.BlockSpec((NC, rb, D_FEAT), lambda i: (0, i, 0)),
        ],
        out_specs=pl.BlockSpec((rb, D_FEAT), lambda i: (i, 0)),
        out_shape=jax.ShapeDtypeStruct((N_NODES, D_FEAT), jnp.float32),
    )(b, partials)

    return out[None]


# speedup vs baseline: 1.1074x; 1.1074x over previous
import functools

import jax
import jax.numpy as jnp
from jax import lax
from jax.experimental import pallas as pl
from jax.experimental.pallas import tpu as pltpu
from jax.experimental.pallas import tpu_sc as plsc

N_NODES = 10000
D_FEAT = 128
N_EDGES = 320000

NC = 2
NS = 16
NW = NC * NS
L = 16

EPW = N_EDGES // NW
CH = 80
NCHUNK = EPW // CH

DRAIN = 624
DRAIN_TAIL = N_NODES - NS * DRAIN


def _combine_body(w_ref, x_ref, h_ref, o_ref):
    o_ref[...] = x_ref[...] * w_ref[0] + h_ref[...] * w_ref[1]


def _finish_body(b_ref, p_ref, o_ref):
    s = p_ref[0] + p_ref[1] + b_ref[0]
    o_ref[...] = jax.nn.sigmoid(s)


def _sc_body(y_hbm, src_hbm, dst_hbm, w_hbm, zeros_hbm, out_hbm,
             src_v, dst_v, w_v, rows_v, agg_sh, sem):
    cid = lax.axis_index("c")
    sid = lax.axis_index("s")
    wid = cid * NS + sid

    @pl.when(sid == 0)
    def _():
        pltpu.sync_copy(zeros_hbm, agg_sh)

    plsc.subcore_barrier()

    def chunk_body(c, carry):
        base = wid * EPW + c * CH
        pltpu.sync_copy(src_hbm.at[pl.ds(base, CH)], src_v)
        pltpu.sync_copy(dst_hbm.at[pl.ds(base, CH)], dst_v)
        pltpu.sync_copy(w_hbm.at[pl.ds(base, CH)], w_v)
        pltpu.async_copy(y_hbm.at[src_v], rows_v, sem).wait()

        def edge_body(i, carry2):
            wspl = plsc.load_gather(w_v, [jnp.full((L,), i, jnp.int32)])
            for j in range(D_FEAT // L):
                rows_v[i, pl.ds(j * L, L)] = rows_v[i, pl.ds(j * L, L)] * wspl
            return carry2

        lax.fori_loop(0, CH, edge_body, 0)
        pltpu.sync_copy(rows_v, agg_sh.at[dst_v], add=True)
        return carry

    lax.fori_loop(0, NCHUNK, chunk_body, 0)

    plsc.subcore_barrier()
    pltpu.sync_copy(agg_sh.at[pl.ds(sid * DRAIN, DRAIN)],
                    out_hbm.at[cid, pl.ds(sid * DRAIN, DRAIN)])

    @pl.when(sid == 0)
    def _():
        pltpu.sync_copy(agg_sh.at[pl.ds(NS * DRAIN, DRAIN_TAIL)],
                        out_hbm.at[cid, pl.ds(NS * DRAIN, DRAIN_TAIL)])


_sc_scatter = functools.partial(
    pl.kernel,
    out_type=jax.ShapeDtypeStruct((NC, N_NODES, D_FEAT), jnp.float32),
    mesh=plsc.VectorSubcoreMesh(core_axis_name="c", subcore_axis_name="s",
                                num_cores=NC, num_subcores=NS),
    compiler_params=pltpu.CompilerParams(needs_layout_passes=False),
    scratch_types=[
        pltpu.VMEM((CH,), jnp.int32),
        pltpu.VMEM((CH,), jnp.int32),
        pltpu.VMEM((CH,), jnp.float32),
        pltpu.VMEM((CH, D_FEAT), jnp.float32),
        pltpu.VMEM_SHARED((N_NODES, D_FEAT), jnp.float32),
        pltpu.SemaphoreType.DMA,
    ],
)(_sc_body)


@jax.jit
def kernel(x, hint_matrix, edge_index, edge_weights, W, b):
    xs = x[0]
    hs = hint_matrix[0]
    wv = W[:, 0]
    src = edge_index[0]
    dst = edge_index[1]

    rb = 1000
    y = pl.pallas_call(
        _combine_body,
        grid=(N_NODES // rb,),
        in_specs=[
            pl.BlockSpec(memory_space=pltpu.SMEM),
            pl.BlockSpec((rb, D_FEAT), lambda i: (i, 0)),
            pl.BlockSpec((rb, D_FEAT), lambda i: (i, 0)),
        ],
        out_specs=pl.BlockSpec((rb, D_FEAT), lambda i: (i, 0)),
        out_shape=jax.ShapeDtypeStruct((N_NODES, D_FEAT), jnp.float32),
    )(wv, xs, hs)

    zeros = jnp.zeros((N_NODES, D_FEAT), jnp.float32)
    partials = _sc_scatter(y, src, dst, edge_weights, zeros)

    out = pl.pallas_call(
        _finish_body,
        grid=(N_NODES // rb,),
        in_specs=[
            pl.BlockSpec(memory_space=pltpu.SMEM),
            pl.BlockSpec((NC, rb, D_FEAT), lambda i: (0, i, 0)),
        ],
        out_specs=pl.BlockSpec((rb, D_FEAT), lambda i: (i, 0)),
        out_shape=jax.ShapeDtypeStruct((N_NODES, D_FEAT), jnp.float32),
    )(b, partials)

    return out[None]
